# Initial kernel scaffold; baseline (speedup 1.0000x reference)
import jax
import jax.numpy as jnp
from jax.experimental import pallas as pl


def kernel(x, edge_index, edge_attr, batch, We, be, W1, b1, Wc, bc):
    raise NotImplementedError("write your pallas kernel here")


# speedup vs baseline: 2.9297x; 2.9297x over previous
import functools

import jax
import jax.numpy as jnp
from jax import lax
from jax.experimental import pallas as pl
from jax.experimental.pallas import tpu as pltpu
from jax.experimental.pallas import tpu_sc as plsc

N = 10000
E = 320000
D = 128
DE = 16
H = 128
G = 128

NC = 2
NS = 16
NW = NC * NS
EPW = E // NW
C = 80
NCHUNK = EPW // C
RPS = 624
RPS_LAST = N - RPS * (NS - 1)


def _emb_body(ea_ref, we_ref, be_ref, out_ref):
    out_ref[...] = (
        jnp.dot(ea_ref[...], we_ref[...], preferred_element_type=jnp.float32)
        + be_ref[...]
    )


def _edge_emb(edge_attr, We, be2):
    BE = 8000
    return pl.pallas_call(
        _emb_body,
        grid=(E // BE,),
        in_specs=[
            pl.BlockSpec((BE, DE), lambda i: (i, 0)),
            pl.BlockSpec((DE, D), lambda i: (0, 0)),
            pl.BlockSpec((1, D), lambda i: (0, 0)),
        ],
        out_specs=pl.BlockSpec((BE, D), lambda i: (i, 0)),
        out_shape=jax.ShapeDtypeStruct((E, D), jnp.float32),
    )(edge_attr, We, be2)


def _sc_body(src_hbm, dst_hbm, x_hbm, emb_hbm, zeros_hbm, out_hbm,
             sidx, didx, xrows, erows, agg_sh, sem):
    cid = lax.axis_index("c")
    sid = lax.axis_index("s")
    wid = sid * NC + cid

    @pl.when(sid < NS - 1)
    def _():
        pltpu.sync_copy(zeros_hbm.at[pl.ds(0, RPS)],
                        agg_sh.at[pl.ds(sid * RPS, RPS)])

    @pl.when(sid == NS - 1)
    def _():
        pltpu.sync_copy(zeros_hbm, agg_sh.at[pl.ds(RPS * (NS - 1), RPS_LAST)])

    plsc.subcore_barrier()

    base0 = wid * EPW

    def chunk(i, carry):
        base = base0 + i * C
        pltpu.sync_copy(src_hbm.at[pl.ds(base, C)], sidx)
        pltpu.sync_copy(dst_hbm.at[pl.ds(base, C)], didx)
        gcp = pltpu.async_copy(x_hbm.at[sidx], xrows, sem)
        pltpu.sync_copy(emb_hbm.at[pl.ds(base, C)], erows)
        gcp.wait()

        def row(e, c2):
            for k in range(D // 16):
                sl = pl.ds(k * 16, 16)
                erows[e, sl] = jnp.maximum(xrows[e, sl] + erows[e, sl], 0.0)
            return c2

        lax.fori_loop(0, C, row, 0)
        pltpu.sync_copy(erows, agg_sh.at[didx], add=True)
        return carry

    lax.fori_loop(0, NCHUNK, chunk, 0)
    plsc.subcore_barrier()

    @pl.when(sid < NS - 1)
    def _():
        pltpu.sync_copy(agg_sh.at[pl.ds(sid * RPS, RPS)],
                        out_hbm.at[cid, pl.ds(sid * RPS, RPS)])

    @pl.when(sid == NS - 1)
    def _():
        pltpu.sync_copy(agg_sh.at[pl.ds(RPS * (NS - 1), RPS_LAST)],
                        out_hbm.at[cid, pl.ds(RPS * (NS - 1), RPS_LAST)])


def _sc_aggregate(src, dst, x, emb, zeros):
    mesh = plsc.VectorSubcoreMesh(core_axis_name="c", subcore_axis_name="s")
    k = pl.kernel(
        _sc_body,
        out_type=jax.ShapeDtypeStruct((NC, N, D), jnp.float32),
        mesh=mesh,
        scratch_types=[
            pltpu.VMEM((C,), jnp.int32),
            pltpu.VMEM((C,), jnp.int32),
            pltpu.VMEM((C, D), jnp.float32),
            pltpu.VMEM((C, D), jnp.float32),
            pltpu.VMEM_SHARED((N, D), jnp.float32),
            pltpu.SemaphoreType.DMA,
        ],
    )
    return k(src, dst, x, emb, zeros)


def _mlp_body(x_ref, agg_ref, batch_ref, w1_ref, b1_ref, wc_ref, bc_ref,
              out_ref):
    xa = x_ref[...] + agg_ref[0] + agg_ref[1]
    h = jnp.maximum(
        jnp.dot(xa, w1_ref[...], preferred_element_type=jnp.float32)
        + b1_ref[...],
        0.0,
    )
    gids = lax.broadcasted_iota(jnp.int32, (G, N), 0)
    sel = (batch_ref[...] == gids).astype(jnp.float32)
    cnt = jnp.maximum(jnp.sum(sel, axis=1, keepdims=True), 1.0)
    pooled = jnp.dot(sel, h, preferred_element_type=jnp.float32) / cnt
    out_ref[...] = (
        jnp.dot(pooled, wc_ref[...], preferred_element_type=jnp.float32)
        + bc_ref[...]
    )


def _mlp_pool(x, agg, batch2, W1, b12, Wc, bc2):
    return pl.pallas_call(
        _mlp_body,
        out_shape=jax.ShapeDtypeStruct((G, 2), jnp.float32),
    )(x, agg, batch2, W1, b12, Wc, bc2)


def kernel(x, edge_index, edge_attr, batch, We, be, W1, b1, Wc, bc):
    src = edge_index[0]
    dst = edge_index[1]
    emb = _edge_emb(edge_attr, We.astype(jnp.float32), be.reshape(1, D))
    zeros = jnp.zeros((RPS_LAST, D), jnp.float32)
    agg = _sc_aggregate(src, dst, x, emb, zeros)
    return _mlp_pool(x, agg, batch.reshape(1, N), W1, b1.reshape(1, H),
                     Wc, bc.reshape(1, 2))


# speedup vs baseline: 4.3050x; 1.4694x over previous
import functools

import jax
import jax.numpy as jnp
from jax import lax
from jax.experimental import pallas as pl
from jax.experimental.pallas import tpu as pltpu
from jax.experimental.pallas import tpu_sc as plsc

N = 10000
E = 320000
D = 128
DE = 16
H = 128
G = 128

NC = 2
NS = 16
NW = NC * NS
EPW = E // NW
C = 80
NCHUNK = EPW // C
RPS = 624
RPS_LAST = N - RPS * (NS - 1)


def _emb_body(ea_ref, we_ref, be_ref, out_ref):
    out_ref[...] = (
        jnp.dot(ea_ref[...], we_ref[...], preferred_element_type=jnp.float32)
        + be_ref[...]
    )


def _edge_emb(edge_attr, We, be2):
    BE = 8000
    return pl.pallas_call(
        _emb_body,
        grid=(E // BE,),
        in_specs=[
            pl.BlockSpec((BE, DE), lambda i: (i, 0)),
            pl.BlockSpec((DE, D), lambda i: (0, 0)),
            pl.BlockSpec((1, D), lambda i: (0, 0)),
        ],
        out_specs=pl.BlockSpec((BE, D), lambda i: (i, 0)),
        out_shape=jax.ShapeDtypeStruct((E, D), jnp.float32),
    )(edge_attr, We, be2)


NBUF = 2


def _sc_body(eidx_hbm, x_hbm, emb_hbm, zeros_hbm, out_hbm,
             ib, xb, eb, agg_shared, *sems):
    isems = sems[0:NBUF]
    gsems = sems[NBUF:2 * NBUF]
    esems = sems[2 * NBUF:3 * NBUF]
    cid = lax.axis_index("c")
    sid = lax.axis_index("s")
    wid = sid * NC + cid
    base0 = wid * EPW

    @pl.when(sid < NS - 1)
    def _():
        pltpu.sync_copy(zeros_hbm.at[pl.ds(0, RPS)],
                        agg_shared.at[pl.ds(sid * RPS, RPS)])

    @pl.when(sid == NS - 1)
    def _():
        pltpu.sync_copy(zeros_hbm,
                        agg_shared.at[pl.ds(RPS * (NS - 1), RPS_LAST)])

    plsc.subcore_barrier()

    def idx_start(k, b):
        pltpu.async_copy(eidx_hbm.at[wid, k], ib.at[b], isems[b])

    def idx_wait(k, b):
        pltpu.make_async_copy(eidx_hbm.at[wid, k], ib.at[b],
                              isems[b]).wait()

    def inputs_start(k, b):
        pltpu.async_copy(x_hbm.at[ib.at[b, 0]], xb.at[b], gsems[b])
        pltpu.async_copy(emb_hbm.at[pl.ds(base0 + k * C, C)], eb.at[b],
                         esems[b])

    def inputs_wait(k, b):
        pltpu.make_async_copy(x_hbm.at[ib.at[b, 0]], xb.at[b],
                              gsems[b]).wait()
        pltpu.make_async_copy(emb_hbm.at[pl.ds(base0 + k * C, C)], eb.at[b],
                              esems[b]).wait()

    def compute_scatter(k, b):
        inputs_wait(k, b)

        def row(e, c2):
            for kk in range(D // 16):
                sl = pl.ds(kk * 16, 16)
                xb[b, e, sl] = jnp.maximum(xb[b, e, sl] + eb[b, e, sl], 0.0)
            return c2

        lax.fori_loop(0, C, row, 0)
        pltpu.sync_copy(xb.at[b], agg_shared.at[ib.at[b, 1]], add=True)

    idx_start(0, 0)
    idx_start(1, 1)
    idx_wait(0, 0)
    inputs_start(0, 0)
    idx_wait(1, 1)
    inputs_start(1, 1)
    compute_scatter(0, 0)
    idx_start(2, 0)

    def jbody(j, carry):
        for t in range(2):
            k = 2 * j + 1 + t
            b = (t + 1) % 2
            idx_wait(k + 1, b ^ 1)
            inputs_start(k + 1, b ^ 1)
            compute_scatter(k, b)
            idx_start(k + 2, b)
        return carry

    lax.fori_loop(0, (NCHUNK - 3) // 2, jbody, 0)

    idx_wait(NCHUNK - 1, 0)
    inputs_start(NCHUNK - 1, 0)
    compute_scatter(NCHUNK - 2, 1)
    compute_scatter(NCHUNK - 1, 0)
    plsc.subcore_barrier()

    @pl.when(sid < NS - 1)
    def _():
        pltpu.sync_copy(agg_shared.at[pl.ds(sid * RPS, RPS)],
                        out_hbm.at[cid, pl.ds(sid * RPS, RPS)])

    @pl.when(sid == NS - 1)
    def _():
        pltpu.sync_copy(agg_shared.at[pl.ds(RPS * (NS - 1), RPS_LAST)],
                        out_hbm.at[cid, pl.ds(RPS * (NS - 1), RPS_LAST)])


def _sc_aggregate(eidx, x, emb, zeros):
    mesh = plsc.VectorSubcoreMesh(core_axis_name="c", subcore_axis_name="s")
    k = pl.kernel(
        _sc_body,
        out_type=jax.ShapeDtypeStruct((NC, N, D), jnp.float32),
        mesh=mesh,
        scratch_types=[
            pltpu.VMEM((NBUF, 2, C), jnp.int32),
            pltpu.VMEM((NBUF, C, D), jnp.float32),
            pltpu.VMEM((NBUF, C, D), jnp.float32),
            pltpu.VMEM_SHARED((N, D), jnp.float32),
        ] + [pltpu.SemaphoreType.DMA] * (3 * NBUF),
    )
    return k(eidx, x, emb, zeros)


def _mlp_body(x_ref, agg_ref, batch_ref, w1_ref, b1_ref, wc_ref, bc_ref,
              out_ref):
    xa = x_ref[...] + agg_ref[0] + agg_ref[1]
    h = jnp.maximum(
        jnp.dot(xa, w1_ref[...], preferred_element_type=jnp.float32)
        + b1_ref[...],
        0.0,
    )
    gids = lax.broadcasted_iota(jnp.int32, (G, N), 0)
    sel = (batch_ref[...] == gids).astype(jnp.float32)
    cnt = jnp.maximum(jnp.sum(sel, axis=1, keepdims=True), 1.0)
    pooled = jnp.dot(sel, h, preferred_element_type=jnp.float32) / cnt
    out_ref[...] = (
        jnp.dot(pooled, wc_ref[...], preferred_element_type=jnp.float32)
        + bc_ref[...]
    )


def _mlp_pool(x, agg, batch2, W1, b12, Wc, bc2):
    return pl.pallas_call(
        _mlp_body,
        out_shape=jax.ShapeDtypeStruct((G, 2), jnp.float32),
    )(x, agg, batch2, W1, b12, Wc, bc2)


def kernel(x, edge_index, edge_attr, batch, We, be, W1, b1, Wc, bc):
    eidx = edge_index.reshape(2, NW, NCHUNK, C).transpose(1, 2, 0, 3)
    emb = _edge_emb(edge_attr, We.astype(jnp.float32), be.reshape(1, D))
    zeros = jnp.zeros((RPS_LAST, D), jnp.float32)
    agg = _sc_aggregate(eidx, x, emb, zeros)
    return _mlp_pool(x, agg, batch.reshape(1, N), W1, b1.reshape(1, H),
                     Wc, bc.reshape(1, 2))


# speedup vs baseline: 4.6427x; 1.0784x over previous
import functools

import jax
import jax.numpy as jnp
import numpy as np
from jax import lax
from jax.experimental import pallas as pl
from jax.experimental.pallas import tpu as pltpu
from jax.experimental.pallas import tpu_sc as plsc

_PERM = np.concatenate(
    [np.stack([np.arange(16), np.arange(16, 32)], 1).reshape(32) + 32 * j
     for j in range(4)])

N = 10000
E = 320000
D = 128
DE = 16
H = 128
G = 128

NC = 2
NS = 16
NW = NC * NS
EPW = E // NW
C = 80
NCHUNK = EPW // C
RPS = 624
RPS_LAST = N - RPS * (NS - 1)


def _emb_body(ea_ref, we_ref, be_ref, out_ref):
    out_ref[...] = (
        jnp.dot(ea_ref[...], we_ref[...], preferred_element_type=jnp.float32)
        + be_ref[...]
    )


def _edge_emb(edge_attr, We, be2):
    BE = 8000
    return pl.pallas_call(
        _emb_body,
        grid=(E // BE,),
        in_specs=[
            pl.BlockSpec((BE, DE), lambda i: (i, 0)),
            pl.BlockSpec((DE, D), lambda i: (0, 0)),
            pl.BlockSpec((1, D), lambda i: (0, 0)),
        ],
        out_specs=pl.BlockSpec((BE, D), lambda i: (i, 0)),
        out_shape=jax.ShapeDtypeStruct((E, D), jnp.float32),
    )(edge_attr, We, be2)


NBUF = 2


def _sc_body(eidx_hbm, x_hbm, emb_hbm, zeros_hbm, out_hbm,
             ib, xb, eb, agg_shared, *sems):
    isems = sems[0:4]
    gsems = sems[4:4 + NBUF]
    esems = sems[4 + NBUF:4 + 2 * NBUF]
    ssems = sems[4 + 2 * NBUF:4 + 3 * NBUF]
    cid = lax.axis_index("c")
    sid = lax.axis_index("s")
    wid = sid * NC + cid
    base0 = wid * EPW

    @pl.when(sid < NS - 1)
    def _():
        pltpu.sync_copy(zeros_hbm.at[pl.ds(0, RPS)],
                        agg_shared.at[pl.ds(sid * RPS, RPS)])

    @pl.when(sid == NS - 1)
    def _():
        pltpu.sync_copy(zeros_hbm,
                        agg_shared.at[pl.ds(RPS * (NS - 1), RPS_LAST)])

    plsc.subcore_barrier()

    def idx_start(k, b):
        pltpu.async_copy(eidx_hbm.at[wid, k], ib.at[b], isems[b])

    def idx_wait(k, b):
        pltpu.make_async_copy(eidx_hbm.at[wid, k], ib.at[b],
                              isems[b]).wait()

    def inputs_start(k, b, bi):
        pltpu.async_copy(x_hbm.at[ib.at[bi, 0]], xb.at[b], gsems[b])
        pltpu.async_copy(emb_hbm.at[pl.ds(base0 + k * C, C)], eb.at[b],
                         esems[b])

    def inputs_wait(k, b, bi):
        pltpu.make_async_copy(x_hbm.at[ib.at[bi, 0]], xb.at[b],
                              gsems[b]).wait()
        pltpu.make_async_copy(emb_hbm.at[pl.ds(base0 + k * C, C)], eb.at[b],
                              esems[b]).wait()

    def scatter_wait(k, b, bi):
        pltpu.make_async_copy(xb.at[b], agg_shared.at[ib.at[bi, 1]],
                              ssems[b]).wait()

    def compute_scatter(k, b, bi):
        inputs_wait(k, b, bi)

        @plsc.parallel_loop(0, C, 1, unroll=2)
        def _(e):
            for kk in range(D // 16):
                sl = pl.ds(kk * 16, 16)
                xb[b, e, sl] = jnp.maximum(xb[b, e, sl] + eb[b, e, sl], 0.0)

        pltpu.async_copy(xb.at[b], agg_shared.at[ib.at[bi, 1]], ssems[b],
                         add=True)

    idx_start(0, 0)
    idx_start(1, 1)
    idx_start(2, 2)
    idx_wait(0, 0)
    inputs_start(0, 0, 0)
    idx_wait(1, 1)
    inputs_start(1, 1, 1)
    compute_scatter(0, 0, 0)
    idx_start(3, 3)

    def step(k, t):
        idx_wait(k + 1, (t + 2) % 4)
        scatter_wait(k - 1, t % 2, t % 4)
        inputs_start(k + 1, t % 2, (t + 2) % 4)
        compute_scatter(k, (t + 1) % 2, (t + 1) % 4)
        idx_start(k + 3, t % 4)

    def jbody(j, carry):
        for t in range(4):
            step(4 * j + 1 + t, t)
        return carry

    lax.fori_loop(0, (NCHUNK - 5) // 4, jbody, 0)

    k0 = NCHUNK - 4
    step(k0, 0)
    idx_wait(k0 + 2, 3)
    scatter_wait(k0, 1, 1)
    inputs_start(k0 + 2, 1, 3)
    compute_scatter(k0 + 1, 0, 2)
    idx_wait(k0 + 3, 0)
    scatter_wait(k0 + 1, 0, 2)
    inputs_start(k0 + 3, 0, 0)
    compute_scatter(k0 + 2, 1, 3)
    compute_scatter(k0 + 3, 0, 0)
    scatter_wait(k0 + 2, 1, 3)
    scatter_wait(k0 + 3, 0, 0)
    plsc.subcore_barrier()

    @pl.when(sid < NS - 1)
    def _():
        pltpu.sync_copy(agg_shared.at[pl.ds(sid * RPS, RPS)],
                        out_hbm.at[cid, pl.ds(sid * RPS, RPS)])

    @pl.when(sid == NS - 1)
    def _():
        pltpu.sync_copy(agg_shared.at[pl.ds(RPS * (NS - 1), RPS_LAST)],
                        out_hbm.at[cid, pl.ds(RPS * (NS - 1), RPS_LAST)])


def _sc_aggregate(eidx, x, emb, zeros):
    mesh = plsc.VectorSubcoreMesh(core_axis_name="c", subcore_axis_name="s")
    k = pl.kernel(
        _sc_body,
        out_type=jax.ShapeDtypeStruct((NC, N, D), jnp.float32),
        mesh=mesh,
        scratch_types=[
            pltpu.VMEM((4, 2, C), jnp.int32),
            pltpu.VMEM((NBUF, C, D), jnp.float32),
            pltpu.VMEM((NBUF, C, D), jnp.float32),
            pltpu.VMEM_SHARED((N, D), jnp.float32),
        ] + [pltpu.SemaphoreType.DMA] * (4 + 3 * NBUF),
    )
    return k(eidx, x, emb, zeros)


def _mlp_body(x_ref, agg_ref, batch_ref, w1_ref, b1_ref, wc_ref, bc_ref,
              out_ref):
    xa = x_ref[...] + agg_ref[0] + agg_ref[1]
    h = jnp.maximum(
        jnp.dot(xa, w1_ref[...], preferred_element_type=jnp.float32)
        + b1_ref[...],
        0.0,
    )
    gids = lax.broadcasted_iota(jnp.int32, (G, N), 0)
    sel = (batch_ref[...] == gids).astype(jnp.float32)
    cnt = jnp.maximum(jnp.sum(sel, axis=1, keepdims=True), 1.0)
    pooled = jnp.dot(sel, h, preferred_element_type=jnp.float32) / cnt
    out_ref[...] = (
        jnp.dot(pooled, wc_ref[...], preferred_element_type=jnp.float32)
        + bc_ref[...]
    )


def _mlp_pool(x, agg, batch2, W1, b12, Wc, bc2):
    return pl.pallas_call(
        _mlp_body,
        out_shape=jax.ShapeDtypeStruct((G, 2), jnp.float32),
    )(x, agg, batch2, W1, b12, Wc, bc2)


def kernel(x, edge_index, edge_attr, batch, We, be, W1, b1, Wc, bc):
    eidx = edge_index.reshape(2, NW, NCHUNK, C).transpose(1, 2, 0, 3)
    emb = _edge_emb(edge_attr, We.astype(jnp.float32), be.reshape(1, D))
    zeros = jnp.zeros((RPS_LAST, D), jnp.float32)
    agg = _sc_aggregate(eidx, x, emb, zeros)
    return _mlp_pool(x, agg, batch.reshape(1, N), W1, b1.reshape(1, H),
                     Wc, bc.reshape(1, 2))


# speedup vs baseline: 4.6903x; 1.0103x over previous
import functools

import jax
import jax.numpy as jnp
import numpy as np
from jax import lax
from jax.experimental import pallas as pl
from jax.experimental.pallas import tpu as pltpu
from jax.experimental.pallas import tpu_sc as plsc

def _pack_pairs(v):
    lo = jax.lax.bitcast_convert_type(
        v[..., :64].astype(jnp.bfloat16), jnp.uint16).astype(jnp.uint32)
    hi = jax.lax.bitcast_convert_type(
        v[..., 64:].astype(jnp.bfloat16), jnp.uint16).astype(jnp.uint32)
    return jax.lax.bitcast_convert_type(lo | (hi << 16), jnp.float32)

N = 10000
E = 320000
D = 128
DE = 16
H = 128
G = 128

NC = 2
NS = 16
NW = NC * NS
EPW = E // NW
C = 80
NCHUNK = EPW // C
RPS = 624
RPS_LAST = N - RPS * (NS - 1)


def _emb_body(ea_ref, we_ref, be_ref, out_ref):
    out_ref[...] = _pack_pairs(
        jnp.dot(ea_ref[...], we_ref[...], preferred_element_type=jnp.float32)
        + be_ref[...]
    )


def _edge_emb(edge_attr, We, be2):
    BE = 8000
    return pl.pallas_call(
        _emb_body,
        grid=(E // BE,),
        in_specs=[
            pl.BlockSpec((BE, DE), lambda i: (i, 0)),
            pl.BlockSpec((DE, D), lambda i: (0, 0)),
            pl.BlockSpec((1, D), lambda i: (0, 0)),
        ],
        out_specs=pl.BlockSpec((BE, D // 2), lambda i: (i, 0)),
        out_shape=jax.ShapeDtypeStruct((E, D // 2), jnp.float32),
    )(edge_attr, We, be2)


NBUF = 2


def _sc_body(eidx_hbm, x_hbm, emb_hbm, zeros_hbm, out_hbm,
             ib, xb, eb, agg_shared, *sems):
    isems = sems[0:4]
    gsems = sems[4:4 + NBUF]
    esems = sems[4 + NBUF:4 + 2 * NBUF]
    ssems = sems[4 + 2 * NBUF:4 + 3 * NBUF]
    cid = lax.axis_index("c")
    sid = lax.axis_index("s")
    wid = sid * NC + cid
    base0 = wid * EPW

    @pl.when(sid < NS - 1)
    def _():
        pltpu.sync_copy(zeros_hbm.at[pl.ds(0, RPS)],
                        agg_shared.at[pl.ds(sid * RPS, RPS)])

    @pl.when(sid == NS - 1)
    def _():
        pltpu.sync_copy(zeros_hbm,
                        agg_shared.at[pl.ds(RPS * (NS - 1), RPS_LAST)])

    plsc.subcore_barrier()

    def idx_start(k, b):
        pltpu.async_copy(eidx_hbm.at[wid, k], ib.at[b], isems[b])

    def idx_wait(k, b):
        pltpu.make_async_copy(eidx_hbm.at[wid, k], ib.at[b],
                              isems[b]).wait()

    def inputs_start(k, b, bi):
        pltpu.async_copy(x_hbm.at[ib.at[bi, 0]], xb.at[b], gsems[b])
        pltpu.async_copy(emb_hbm.at[pl.ds(base0 + k * C, C)], eb.at[b],
                         esems[b])

    def inputs_wait(k, b, bi):
        pltpu.make_async_copy(x_hbm.at[ib.at[bi, 0]], xb.at[b],
                              gsems[b]).wait()
        pltpu.make_async_copy(emb_hbm.at[pl.ds(base0 + k * C, C)], eb.at[b],
                              esems[b]).wait()

    def scatter_wait(k, b, bi):
        pltpu.make_async_copy(xb.at[b], agg_shared.at[ib.at[bi, 1]],
                              ssems[b]).wait()

    def compute_scatter(k, b, bi):
        inputs_wait(k, b, bi)

        @plsc.parallel_loop(0, C, 1, unroll=2)
        def _(e):
            for kk in range(D // 32):
                u = jax.lax.bitcast_convert_type(
                    eb[b, e, pl.ds(16 * kk, 16)], jnp.int32)
                elo = jax.lax.bitcast_convert_type(
                    jax.lax.shift_left(u, jnp.int32(16)), jnp.float32)
                ehi = jax.lax.bitcast_convert_type(
                    jax.lax.bitwise_and(u, jnp.int32(-65536)), jnp.float32)
                slo = pl.ds(16 * kk, 16)
                shi = pl.ds(64 + 16 * kk, 16)
                xb[b, e, slo] = jnp.maximum(xb[b, e, slo] + elo, 0.0)
                xb[b, e, shi] = jnp.maximum(xb[b, e, shi] + ehi, 0.0)

        pltpu.async_copy(xb.at[b], agg_shared.at[ib.at[bi, 1]], ssems[b],
                         add=True)

    idx_start(0, 0)
    idx_start(1, 1)
    idx_start(2, 2)
    idx_wait(0, 0)
    inputs_start(0, 0, 0)
    idx_wait(1, 1)
    inputs_start(1, 1, 1)
    compute_scatter(0, 0, 0)
    idx_start(3, 3)

    def step(k, t):
        idx_wait(k + 1, (t + 2) % 4)
        scatter_wait(k - 1, t % 2, t % 4)
        inputs_start(k + 1, t % 2, (t + 2) % 4)
        compute_scatter(k, (t + 1) % 2, (t + 1) % 4)
        idx_start(k + 3, t % 4)

    def jbody(j, carry):
        for t in range(4):
            step(4 * j + 1 + t, t)
        return carry

    lax.fori_loop(0, (NCHUNK - 5) // 4, jbody, 0)

    k0 = NCHUNK - 4
    step(k0, 0)
    idx_wait(k0 + 2, 3)
    scatter_wait(k0, 1, 1)
    inputs_start(k0 + 2, 1, 3)
    compute_scatter(k0 + 1, 0, 2)
    idx_wait(k0 + 3, 0)
    scatter_wait(k0 + 1, 0, 2)
    inputs_start(k0 + 3, 0, 0)
    compute_scatter(k0 + 2, 1, 3)
    compute_scatter(k0 + 3, 0, 0)
    scatter_wait(k0 + 2, 1, 3)
    scatter_wait(k0 + 3, 0, 0)
    plsc.subcore_barrier()

    @pl.when(sid < NS - 1)
    def _():
        pltpu.sync_copy(agg_shared.at[pl.ds(sid * RPS, RPS)],
                        out_hbm.at[cid, pl.ds(sid * RPS, RPS)])

    @pl.when(sid == NS - 1)
    def _():
        pltpu.sync_copy(agg_shared.at[pl.ds(RPS * (NS - 1), RPS_LAST)],
                        out_hbm.at[cid, pl.ds(RPS * (NS - 1), RPS_LAST)])


def _sc_aggregate(eidx, x, emb, zeros):
    mesh = plsc.VectorSubcoreMesh(core_axis_name="c", subcore_axis_name="s")
    k = pl.kernel(
        _sc_body,
        out_type=jax.ShapeDtypeStruct((NC, N, D), jnp.float32),
        mesh=mesh,
        scratch_types=[
            pltpu.VMEM((4, 2, C), jnp.int32),
            pltpu.VMEM((NBUF, C, D), jnp.float32),
            pltpu.VMEM((NBUF, C, D // 2), jnp.float32),
            pltpu.VMEM_SHARED((N, D), jnp.float32),
        ] + [pltpu.SemaphoreType.DMA] * (4 + 3 * NBUF),
    )
    return k(eidx, x, emb, zeros)


def _mlp_body(x_ref, agg_ref, batch_ref, w1_ref, b1_ref, wc_ref, bc_ref,
              out_ref):
    xa = x_ref[...] + agg_ref[0] + agg_ref[1]
    h = jnp.maximum(
        jnp.dot(xa, w1_ref[...], preferred_element_type=jnp.float32)
        + b1_ref[...],
        0.0,
    )
    gids = lax.broadcasted_iota(jnp.int32, (G, N), 0)
    sel = (batch_ref[...] == gids).astype(jnp.float32)
    cnt = jnp.maximum(jnp.sum(sel, axis=1, keepdims=True), 1.0)
    pooled = jnp.dot(sel, h, preferred_element_type=jnp.float32) / cnt
    out_ref[...] = (
        jnp.dot(pooled, wc_ref[...], preferred_element_type=jnp.float32)
        + bc_ref[...]
    )


def _mlp_pool(x, agg, batch2, W1, b12, Wc, bc2):
    return pl.pallas_call(
        _mlp_body,
        out_shape=jax.ShapeDtypeStruct((G, 2), jnp.float32),
    )(x, agg, batch2, W1, b12, Wc, bc2)


def kernel(x, edge_index, edge_attr, batch, We, be, W1, b1, Wc, bc):
    eidx = edge_index.reshape(2, NW, NCHUNK, C).transpose(1, 2, 0, 3)
    emb = _edge_emb(edge_attr, We.astype(jnp.float32), be.reshape(1, D))
    zeros = jnp.zeros((RPS_LAST, D), jnp.float32)
    agg = _sc_aggregate(eidx, x, emb, zeros)
    return _mlp_pool(x, agg, batch.reshape(1, N), W1, b1.reshape(1, H),
                     Wc, bc.reshape(1, 2))
